# baseline (speedup 1.0000x reference)
import functools

import jax
import jax.numpy as jnp
from jax import lax
from jax.experimental import pallas as pl
from jax.experimental.pallas import tpu as pltpu
from jax.experimental.pallas import tpu_sc as plsc

B = 16384
D = 32
V = 1000000
LB = 512
KROWS = 250112
TH = 999936
PASS = 128


def _pack(ut_t, it_t):
    nblk = (V + LB - 1) // LB

    def body(u_ref, i_ref, uo_ref, io_ref):
        u = u_ref[...]
        i = i_ref[...]
        uo_ref[...] = jnp.transpose(jnp.reshape(u, (D, LB // 4, 4)), (1, 2, 0)).reshape(LB // 4, 4 * D)
        io_ref[...] = jnp.transpose(jnp.reshape(i, (D, LB // 4, 4)), (1, 2, 0)).reshape(LB // 4, 4 * D)

    return pl.pallas_call(
        body,
        grid=(nblk,),
        in_specs=[
            pl.BlockSpec((D, LB), lambda g: (0, g)),
            pl.BlockSpec((D, LB), lambda g: (0, g)),
        ],
        out_specs=[
            pl.BlockSpec((LB // 4, 4 * D), lambda g: (g, 0)),
            pl.BlockSpec((LB // 4, 4 * D), lambda g: (g, 0)),
        ],
        out_shape=[
            jax.ShapeDtypeStruct((KROWS, 128), jnp.float32),
            jax.ShapeDtypeStruct((KROWS, 128), jnp.float32),
        ],
    )(ut_t, it_t)


def _gather_dot(uid, iid, upack, ipack, tailu, taili):
    info = plsc.get_sparse_core_info()
    nc, ns = info.num_cores, info.num_subcores
    nw = nc * ns
    bpw = B // nw
    npass = bpw // PASS
    mesh = plsc.VectorSubcoreMesh(core_axis_name="c", subcore_axis_name="s")

    @functools.partial(
        pl.kernel,
        mesh=mesh,
        compiler_params=pltpu.CompilerParams(
            use_tc_tiling_on_sc=True, needs_layout_passes=False),
        out_type=jax.ShapeDtypeStruct((B,), jnp.float32),
        scratch_types=[
            pltpu.VMEM((bpw,), jnp.int32),
            pltpu.VMEM((bpw,), jnp.int32),
            pltpu.VMEM((PASS,), jnp.int32),
            pltpu.VMEM((PASS,), jnp.int32),
            pltpu.VMEM((PASS, 128), jnp.float32),
            pltpu.VMEM((PASS, 128), jnp.float32),
            pltpu.VMEM((D, 128), jnp.float32),
            pltpu.VMEM((D, 128), jnp.float32),
            pltpu.VMEM((bpw,), jnp.float32),
            pltpu.SemaphoreType.DMA,
            pltpu.SemaphoreType.DMA,
        ],
    )
    def kb(uid_hbm, iid_hbm, up_hbm, ip_hbm, tailu_hbm, taili_hbm, out_hbm,
           uid_v, iid_v, ku_v, ki_v, du_v, di_v, tu_v, ti_v, out_v, semu, semi):
        wid = lax.axis_index("s") * nc + lax.axis_index("c")
        base = wid * bpw
        pltpu.sync_copy(uid_hbm.at[pl.ds(base, bpw)], uid_v)
        pltpu.sync_copy(iid_hbm.at[pl.ds(base, bpw)], iid_v)
        pltpu.sync_copy(tailu_hbm, tu_v)
        pltpu.sync_copy(taili_hbm, ti_v)

        def pass_body(p, _):
            p0 = p * PASS

            def krows(c, _):
                u = uid_v[pl.ds(p0 + c * 16, 16)]
                i = iid_v[pl.ds(p0 + c * 16, 16)]
                ku_v[pl.ds(c * 16, 16)] = u >> 2
                ki_v[pl.ds(c * 16, 16)] = i >> 2
                return 0

            lax.fori_loop(0, PASS // 16, krows, 0)

            cu = pltpu.async_copy(up_hbm.at[ku_v], du_v, semu)
            ci = pltpu.async_copy(ip_hbm.at[ki_v], di_v, semi)
            cu.wait()
            ci.wait()

            def dot(g, _):
                rows = g * 16 + lax.iota(jnp.int32, 16)
                u16 = uid_v[pl.ds(p0 + g * 16, 16)]
                i16 = iid_v[pl.ds(p0 + g * 16, 16)]
                ucol0 = (u16 & 3) * D
                icol0 = (i16 & 3) * D
                umask = u16 >= TH
                imask = i16 >= TH
                uti = jnp.maximum(u16 - TH, 0)
                iti = jnp.maximum(i16 - TH, 0)
                acc = jnp.zeros((16,), jnp.float32)
                for d in range(D):
                    dvec = jnp.full((16,), d, jnp.int32)
                    u = plsc.load_gather(du_v, [rows, ucol0 + d])
                    i = plsc.load_gather(di_v, [rows, icol0 + d])
                    tu = plsc.load_gather(tu_v, [dvec, uti])
                    ti = plsc.load_gather(ti_v, [dvec, iti])
                    u = jnp.where(umask, tu, u)
                    i = jnp.where(imask, ti, i)
                    acc += u * i
                out_v[pl.ds(p0 + g * 16, 16)] = acc
                return 0

            lax.fori_loop(0, PASS // 16, dot, 0)
            return 0

        lax.fori_loop(0, npass, pass_body, 0)
        pltpu.sync_copy(out_v, out_hbm.at[pl.ds(base, bpw)])

    return kb(uid, iid, upack, ipack, tailu, taili)


def kernel(input_userID, input_itemID, user_table, item_table):
    uid = input_userID.astype(jnp.int32)
    iid = input_itemID.astype(jnp.int32)
    tailu = jnp.pad(user_table[TH:].T, ((0, 0), (0, 128 - (V - TH))))
    taili = jnp.pad(item_table[TH:].T, ((0, 0), (0, 128 - (V - TH))))
    upack, ipack = _pack(user_table.T, item_table.T)
    return _gather_dot(uid, iid, upack, ipack, tailu, taili)


# speedup vs baseline: 6.0337x; 6.0337x over previous
import functools

import jax
import jax.numpy as jnp
from jax import lax
from jax.experimental import pallas as pl
from jax.experimental.pallas import tpu as pltpu
from jax.experimental.pallas import tpu_sc as plsc

B = 16384
D = 32
V = 1000000


def _sc_fused(uid, iid, ut, it):
    info = plsc.get_sparse_core_info()
    nc, ns = info.num_cores, info.num_subcores
    nw = nc * ns
    bpw = B // nw
    ng = bpw // 16
    mesh = plsc.VectorSubcoreMesh(core_axis_name="c", subcore_axis_name="s")

    @functools.partial(
        pl.kernel,
        mesh=mesh,
        compiler_params=pltpu.CompilerParams(
            use_tc_tiling_on_sc=False, needs_layout_passes=False),
        out_type=jax.ShapeDtypeStruct((B,), jnp.float32),
        scratch_types=[
            pltpu.VMEM((bpw,), jnp.int32),
            pltpu.VMEM((bpw,), jnp.int32),
            pltpu.VMEM((bpw, D), jnp.float32),
            pltpu.VMEM((bpw, D), jnp.float32),
            pltpu.VMEM((bpw,), jnp.float32),
            pltpu.SemaphoreType.DMA,
            pltpu.SemaphoreType.DMA,
        ],
    )
    def k(uid_hbm, iid_hbm, ut_hbm, it_hbm, out_hbm,
          uid_v, iid_v, du_v, di_v, out_v, semu, semi):
        wid = lax.axis_index("s") * nc + lax.axis_index("c")
        base = wid * bpw
        pltpu.sync_copy(uid_hbm.at[pl.ds(base, bpw)], uid_v)
        pltpu.sync_copy(iid_hbm.at[pl.ds(base, bpw)], iid_v)

        cu = pltpu.async_copy(ut_hbm.at[uid_v], du_v, semu)
        ci = pltpu.async_copy(it_hbm.at[iid_v], di_v, semi)
        cu.wait()
        ci.wait()

        def dot(g, _):
            rows = g * 16 + lax.iota(jnp.int32, 16)
            acc = jnp.zeros((16,), jnp.float32)
            for d in range(D):
                dvec = jnp.full((16,), d, jnp.int32)
                u = plsc.load_gather(du_v, [rows, dvec])
                i = plsc.load_gather(di_v, [rows, dvec])
                acc += u * i
            out_v[pl.ds(g * 16, 16)] = acc
            return 0

        lax.fori_loop(0, ng, dot, 0)
        pltpu.sync_copy(out_v, out_hbm.at[pl.ds(base, bpw)])

    return k(uid, iid, ut, it)


def kernel(input_userID, input_itemID, user_table, item_table):
    uid = input_userID.astype(jnp.int32)
    iid = input_itemID.astype(jnp.int32)
    return _sc_fused(uid, iid, user_table, item_table)
